# baseline (speedup 1.0000x reference)
import functools

import jax
import jax.numpy as jnp
from jax import lax
from jax.experimental import pallas as pl
from jax.experimental.pallas import tpu as pltpu
from jax.experimental.pallas import tpu_sc as plsc

_N_MARGIN = 0.3
_SENTINEL = -2.0
_LANES = 16


def _sc_geometry():
    try:
        info = plsc.get_sparse_core_info()
        return info.num_cores, info.num_subcores
    except Exception:
        return 2, 16


@functools.lru_cache(maxsize=None)
def _make_sc_kernel(Bn, Dn, NNEG):
    NC, NS = _sc_geometry()
    NW = NC * NS
    RW = Bn // NW
    NV = RW * NNEG
    CH = 128
    NCH = NV // CH
    RPC = CH // NNEG
    KD = Dn // _LANES
    mesh = plsc.VectorSubcoreMesh(core_axis_name="c", subcore_axis_name="s")

    assert NNEG == 2 * _LANES and RW % _LANES == 0

    @functools.partial(
        pl.kernel,
        out_type=(
            jax.ShapeDtypeStruct((Bn, NNEG), jnp.float32),
            jax.ShapeDtypeStruct((Bn,), jnp.float32),
            jax.ShapeDtypeStruct((Bn,), jnp.float32),
        ),
        mesh=mesh,
        compiler_params=pltpu.CompilerParams(needs_layout_passes=False),
        scratch_types=[
            pltpu.VMEM((NV,), jnp.int32),
            pltpu.VMEM((NV,), jnp.int32),
            pltpu.VMEM((RW,), jnp.int32),
            pltpu.VMEM((RW, Dn), jnp.float32),
            pltpu.VMEM((RW, Dn), jnp.float32),
            pltpu.VMEM((CH, Dn), jnp.float32),
            pltpu.VMEM((CH, Dn), jnp.float32),
            pltpu.VMEM((RW, NNEG), jnp.float32),
            pltpu.VMEM((RW,), jnp.float32),
            pltpu.VMEM((RW,), jnp.float32),
            pltpu.SemaphoreType.DMA,
            pltpu.SemaphoreType.DMA,
        ],
    )
    def sc(neg_hbm, tgt_hbm, alc_hbm, inp_hbm, v_hbm,
           nsims_hbm, dotiv_hbm, ss_hbm,
           np_v, cid_v, tgt_v, vt_v, in_v, vc0_v, vc1_v, ns_v, div_v, ss_v,
           sem0, sem1):
        wid = lax.axis_index("s") * NC + lax.axis_index("c")
        base = wid * RW
        lanes = lax.iota(jnp.int32, _LANES)

        pltpu.sync_copy(neg_hbm.at[pl.ds(base * NNEG, NV)], np_v)
        pltpu.sync_copy(tgt_hbm.at[pl.ds(base, RW)], tgt_v)
        pltpu.sync_copy(inp_hbm.at[pl.ds(base, RW)], in_v)

        waits = []
        for c in range(NCH):
            waits.append(pltpu.async_copy(
                alc_hbm.at[np_v.at[pl.ds(c * CH, CH)]],
                cid_v.at[pl.ds(c * CH, CH)], sem0))
        waits.append(pltpu.async_copy(v_hbm.at[tgt_v], vt_v, sem0))
        for w in waits:
            w.wait()

        bufs = (vc0_v, vc1_v)
        sems = (sem0, sem1)
        cps = [None, None]
        cps[0] = pltpu.async_copy(
            v_hbm.at[cid_v.at[pl.ds(0, CH)]], bufs[0], sems[0])

        zvec = jnp.zeros((_LANES,), jnp.float32)
        for g in range(RW // _LANES):
            def rloop(rr, carry, g=g):
                viv, vss = carry
                r = g * _LANES + rr
                acc_iv = jnp.zeros((_LANES,), jnp.float32)
                acc_ss = jnp.zeros((_LANES,), jnp.float32)
                piv = []
                pss = []
                for k in range(KD):
                    xi = in_v[r, pl.ds(k * _LANES, _LANES)]
                    piv.append(xi * vt_v[r, pl.ds(k * _LANES, _LANES)])
                    pss.append(xi * xi)
                while len(piv) > 1:
                    piv = [piv[i] + piv[i + 1] for i in range(0, len(piv), 2)]
                    pss = [pss[i] + pss[i + 1] for i in range(0, len(pss), 2)]
                tiv = jnp.sum(piv[0])
                tss = jnp.sum(pss[0])
                return (jnp.where(lanes == rr, tiv, viv),
                        jnp.where(lanes == rr, tss, vss))
            viv, vss = lax.fori_loop(0, _LANES, rloop, (zvec, zvec))
            div_v[pl.ds(g * _LANES, _LANES)] = viv
            ss_v[pl.ds(g * _LANES, _LANES)] = vss

        zmask = jnp.zeros((_LANES,), jnp.bool_)
        m15 = lanes == (_LANES - 1)
        for c in range(NCH):
            pb = c % 2
            if c + 1 < NCH:
                cps[(c + 1) % 2] = pltpu.async_copy(
                    v_hbm.at[cid_v.at[pl.ds((c + 1) * CH, CH)]],
                    bufs[(c + 1) % 2], sems[(c + 1) % 2])
            cps[pb].wait()
            vc_v = bufs[pb]

            @plsc.parallel_loop(0, CH, step=1, unroll=2)
            def dots(d, c=c, vc_v=vc_v):
                rl = d // NNEG
                j = d - rl * NNEG
                r = c * RPC + rl
                parts = [vt_v[r, pl.ds(k * _LANES, _LANES)] *
                         vc_v[d, pl.ds(k * _LANES, _LANES)]
                         for k in range(KD)]
                while len(parts) > 1:
                    parts = [parts[i] + parts[i + 1]
                             for i in range(0, len(parts), 2)]
                plsc.store_scatter(
                    ns_v, [jnp.broadcast_to(r, (_LANES,)),
                           jnp.broadcast_to(j, (_LANES,))],
                    plsc.cumsum(parts[0]), mask=m15)

            def rlbody(rl, _, c=c):
                r = c * RPC + rl
                row_c0 = cid_v[pl.ds(r * NNEG, _LANES)]
                row_c1 = cid_v[pl.ds(r * NNEG + _LANES, _LANES)]

                def kloop(k, carry, r=r, row_c0=row_c0, row_c1=row_c1):
                    d0, d1 = carry
                    ckv = plsc.load_gather(
                        cid_v, [jnp.broadcast_to(r * NNEG + k, (_LANES,))])
                    d0 = d0 | ((row_c0 == ckv) & (lanes > k))
                    d1 = d1 | ((row_c1 == ckv) & ((lanes + _LANES) > k))
                    return d0, d1
                dup0, dup1 = lax.fori_loop(0, NNEG, kloop, (zmask, zmask),
                                           unroll=2)
                ns_v[r, pl.ds(0, _LANES)] = jnp.where(
                    dup0, _SENTINEL, ns_v[r, pl.ds(0, _LANES)])
                ns_v[r, pl.ds(_LANES, _LANES)] = jnp.where(
                    dup1, _SENTINEL, ns_v[r, pl.ds(_LANES, _LANES)])
                return 0
            lax.fori_loop(0, RPC, rlbody, 0)

        pltpu.sync_copy(ns_v, nsims_hbm.at[pl.ds(base, RW)])
        pltpu.sync_copy(div_v, dotiv_hbm.at[pl.ds(base, RW)])
        pltpu.sync_copy(ss_v, ss_hbm.at[pl.ds(base, RW)])

    return sc


@functools.lru_cache(maxsize=None)
def _make_matmul(Bn, Dn, Cn):
    CBLK = 1024

    def mm(x_ref, v_ref, o_ref):
        o_ref[...] = lax.dot_general(
            x_ref[...], v_ref[...], (((1,), (1,)), ((), ())),
            preferred_element_type=jnp.float32)

    return pl.pallas_call(
        mm,
        grid=(Cn // CBLK,),
        in_specs=[pl.BlockSpec((Bn, Dn), lambda i: (0, 0)),
                  pl.BlockSpec((CBLK, Dn), lambda i: (i, 0))],
        out_specs=pl.BlockSpec((Bn, CBLK), lambda i: (0, i)),
        out_shape=jax.ShapeDtypeStruct((Bn, Cn), jnp.float32),
    )


@functools.lru_cache(maxsize=None)
def _make_finish(Bn, NNEG):
    def fin(ns_ref, div_ref, ss_ref, o_ref):
        ns = ns_ref[...]
        nthr = div_ref[...] * lax.rsqrt(ss_ref[...]) - _N_MARGIN
        hard = (ns > nthr) & (ns < 0.999999)
        sp = jnp.maximum(ns, 0.0) + jnp.log(1.0 + jnp.exp(-jnp.abs(ns)))
        cnt = jnp.sum(hard.astype(jnp.float32))
        tot = jnp.sum(jnp.where(hard, sp, 0.0))
        o_ref[0, 0] = jnp.where(cnt > 0.0, tot / jnp.maximum(cnt, 1.0), 0.0)

    return pl.pallas_call(
        fin,
        in_specs=[pl.BlockSpec((Bn, NNEG), lambda: (0, 0)),
                  pl.BlockSpec((Bn, 1), lambda: (0, 0)),
                  pl.BlockSpec((Bn, 1), lambda: (0, 0))],
        out_specs=pl.BlockSpec(memory_space=pltpu.SMEM),
        out_shape=jax.ShapeDtypeStruct((1, 1), jnp.float32),
    )


def kernel(inputs, targets, pos_pairs, neg_pairs, indexs,
           all_label_to_clusterid, V):
    Bn, Dn = inputs.shape
    Cn = V.shape[0]
    NNEG = neg_pairs.shape[1]

    outputs = _make_matmul(Bn, Dn, Cn)(inputs, V)

    def _i32(x):
        return x if x.dtype == jnp.int32 else x.astype(jnp.int32)

    neg_flat = _i32(neg_pairs.reshape(-1))
    nsims, dotiv, ss = _make_sc_kernel(Bn, Dn, NNEG)(
        neg_flat, _i32(targets), _i32(all_label_to_clusterid), inputs, V)

    loss2 = _make_finish(Bn, NNEG)(
        nsims, dotiv.reshape(Bn, 1), ss.reshape(Bn, 1))
    return (loss2[0, 0], outputs)


# speedup vs baseline: 1.1840x; 1.1840x over previous
import functools

import jax
import jax.numpy as jnp
from jax import lax
from jax.experimental import pallas as pl
from jax.experimental.pallas import tpu as pltpu
from jax.experimental.pallas import tpu_sc as plsc

_N_MARGIN = 0.3
_LANES = 16
_C0 = 0.6931473219368033
_C2 = 0.12499505481299149
_C4 = -0.005181712215175508
_C6 = 0.00030122702988250955


def _sc_geometry():
    try:
        info = plsc.get_sparse_core_info()
        return info.num_cores, info.num_subcores
    except Exception:
        return 2, 16


@functools.lru_cache(maxsize=None)
def _make_sc_kernel(Bn, Dn, NNEG):
    NC, NS = _sc_geometry()
    NW = NC * NS
    RW = Bn // NW
    NV = RW * NNEG
    CH = 128
    NCH = NV // CH
    RPC = CH // NNEG
    KD = Dn // _LANES
    mesh = plsc.VectorSubcoreMesh(core_axis_name="c", subcore_axis_name="s")

    assert NNEG == 2 * _LANES and RW % _LANES == 0

    @functools.partial(
        pl.kernel,
        out_type=(
            jax.ShapeDtypeStruct((NW * _LANES,), jnp.float32),
            jax.ShapeDtypeStruct((NW * _LANES,), jnp.float32),
        ),
        mesh=mesh,
        compiler_params=pltpu.CompilerParams(needs_layout_passes=False),
        scratch_types=[
            pltpu.VMEM((NV,), jnp.int32),
            pltpu.VMEM((NV,), jnp.int32),
            pltpu.VMEM((RW,), jnp.int32),
            pltpu.VMEM((RW, Dn), jnp.float32),
            pltpu.VMEM((RW, Dn), jnp.float32),
            pltpu.VMEM((CH, Dn), jnp.float32),
            pltpu.VMEM((CH, Dn), jnp.float32),
            pltpu.VMEM((RW,), jnp.float32),
            pltpu.VMEM((_LANES,), jnp.float32),
            pltpu.VMEM((_LANES,), jnp.float32),
            pltpu.SemaphoreType.DMA,
            pltpu.SemaphoreType.DMA,
        ],
    )
    def sc(neg_hbm, tgt_hbm, alc_hbm, inp_hbm, v_hbm,
           sums_hbm, cnts_hbm,
           np_v, cid_v, tgt_v, vt_v, in_v, vc0_v, vc1_v, thr_v,
           sa_v, ca_v, sem0, sem1):
        wid = lax.axis_index("s") * NC + lax.axis_index("c")
        base = wid * RW
        lanes = lax.iota(jnp.int32, _LANES)

        pltpu.sync_copy(neg_hbm.at[pl.ds(base * NNEG, NV)], np_v)
        pltpu.sync_copy(tgt_hbm.at[pl.ds(base, RW)], tgt_v)
        pltpu.sync_copy(inp_hbm.at[pl.ds(base, RW)], in_v)

        waits = []
        for c in range(NCH):
            waits.append(pltpu.async_copy(
                alc_hbm.at[np_v.at[pl.ds(c * CH, CH)]],
                cid_v.at[pl.ds(c * CH, CH)], sem0))
        waits.append(pltpu.async_copy(v_hbm.at[tgt_v], vt_v, sem0))
        for w in waits:
            w.wait()

        bufs = (vc0_v, vc1_v)
        sems = (sem0, sem1)
        cps = [None, None]
        cps[0] = pltpu.async_copy(
            v_hbm.at[cid_v.at[pl.ds(0, CH)]], bufs[0], sems[0])

        zvec = jnp.zeros((_LANES,), jnp.float32)
        for g in range(RW // _LANES):
            def rloop(rr, carry, g=g):
                viv, vss = carry
                r = g * _LANES + rr
                piv = []
                pss = []
                for k in range(KD):
                    xi = in_v[r, pl.ds(k * _LANES, _LANES)]
                    piv.append(xi * vt_v[r, pl.ds(k * _LANES, _LANES)])
                    pss.append(xi * xi)
                while len(piv) > 1:
                    piv = [piv[i] + piv[i + 1] for i in range(0, len(piv), 2)]
                    pss = [pss[i] + pss[i + 1] for i in range(0, len(pss), 2)]
                tiv = jnp.sum(piv[0])
                tss = jnp.sum(pss[0])
                return (jnp.where(lanes == rr, tiv, viv),
                        jnp.where(lanes == rr, tss, vss))
            viv, vss = lax.fori_loop(0, _LANES, rloop, (zvec, zvec))
            u = plsc.bitcast(vss, jnp.int32)
            y = plsc.bitcast(jnp.int32(0x5F3759DF) - (u >> 1), jnp.float32)
            for _ in range(3):
                y = y * (1.5 - 0.5 * vss * y * y)
            thr_v[pl.ds(g * _LANES, _LANES)] = viv * y - _N_MARGIN

        zmask = jnp.zeros((_LANES,), jnp.bool_)
        sacc = zvec
        cacc = zvec
        for c in range(NCH):
            pb = c % 2
            if c + 1 < NCH:
                cps[(c + 1) % 2] = pltpu.async_copy(
                    v_hbm.at[cid_v.at[pl.ds((c + 1) * CH, CH)]],
                    bufs[(c + 1) % 2], sems[(c + 1) % 2])
            cps[pb].wait()
            vc_v = bufs[pb]

            def rlbody(rl, carry, c=c, vc_v=vc_v):
                sacc, cacc = carry
                r = c * RPC + rl
                vt_regs = [vt_v[r, pl.ds(k * _LANES, _LANES)]
                           for k in range(KD)]
                row_c0 = cid_v[pl.ds(r * NNEG, _LANES)]
                row_c1 = cid_v[pl.ds(r * NNEG + _LANES, _LANES)]

                def jloop(j, carry2, rl=rl, vt_regs=vt_regs, vc_v=vc_v):
                    v0, v1 = carry2
                    d = rl * NNEG + j
                    parts = [vt_regs[k] * vc_v[d, pl.ds(k * _LANES, _LANES)]
                             for k in range(KD)]
                    while len(parts) > 1:
                        parts = [parts[i] + parts[i + 1]
                                 for i in range(0, len(parts), 2)]
                    tot = jnp.sum(parts[0])
                    return (jnp.where(lanes == j, tot, v0),
                            jnp.where(lanes == (j - _LANES), tot, v1))
                v0, v1 = lax.fori_loop(0, NNEG, jloop, (zvec, zvec),
                                       unroll=2)

                def kloop(k, carry2, r=r, row_c0=row_c0, row_c1=row_c1):
                    d0, d1 = carry2
                    ckv = plsc.load_gather(
                        cid_v, [jnp.broadcast_to(r * NNEG + k, (_LANES,))])
                    d0 = d0 | ((row_c0 == ckv) & (lanes > k))
                    d1 = d1 | ((row_c1 == ckv) & ((lanes + _LANES) > k))
                    return d0, d1
                dup0, dup1 = lax.fori_loop(0, NNEG, kloop, (zmask, zmask),
                                           unroll=2)

                thrv = plsc.load_gather(thr_v, [jnp.broadcast_to(r,
                                                                 (_LANES,))])
                for vv, dd in ((v0, dup0), (v1, dup1)):
                    hard = (~dd) & (vv > thrv) & (vv < 0.999999)
                    t2 = vv * vv
                    sp = 0.5 * vv + (_C0 + t2 * (_C2 + t2 * (_C4 + t2 * _C6)))
                    sacc = sacc + jnp.where(hard, sp, 0.0)
                    cacc = cacc + jnp.where(hard, 1.0, 0.0)
                return sacc, cacc
            sacc, cacc = lax.fori_loop(0, RPC, rlbody, (sacc, cacc))

        sa_v[...] = sacc
        ca_v[...] = cacc
        pltpu.sync_copy(sa_v, sums_hbm.at[pl.ds(wid * _LANES, _LANES)])
        pltpu.sync_copy(ca_v, cnts_hbm.at[pl.ds(wid * _LANES, _LANES)])

    return sc


@functools.lru_cache(maxsize=None)
def _make_matmul(Bn, Dn, Cn):
    CBLK = 1024

    def mm(x_ref, v_ref, o_ref):
        o_ref[...] = lax.dot_general(
            x_ref[...], v_ref[...], (((1,), (1,)), ((), ())),
            preferred_element_type=jnp.float32)

    return pl.pallas_call(
        mm,
        grid=(Cn // CBLK,),
        in_specs=[pl.BlockSpec((Bn, Dn), lambda i: (0, 0)),
                  pl.BlockSpec((CBLK, Dn), lambda i: (i, 0))],
        out_specs=pl.BlockSpec((Bn, CBLK), lambda i: (0, i)),
        out_shape=jax.ShapeDtypeStruct((Bn, Cn), jnp.float32),
    )


def kernel(inputs, targets, pos_pairs, neg_pairs, indexs,
           all_label_to_clusterid, V):
    Bn, Dn = inputs.shape
    Cn = V.shape[0]
    NNEG = neg_pairs.shape[1]

    outputs = _make_matmul(Bn, Dn, Cn)(inputs, V)

    def _i32(x):
        return x if x.dtype == jnp.int32 else x.astype(jnp.int32)

    neg_flat = _i32(neg_pairs.reshape(-1))
    sums, cnts = _make_sc_kernel(Bn, Dn, NNEG)(
        neg_flat, _i32(targets), _i32(all_label_to_clusterid), inputs, V)

    s = jnp.sum(sums)
    c = jnp.sum(cnts)
    loss = jnp.where(c > 0.0, s / jnp.maximum(c, 1.0), jnp.float32(0.0))
    return (loss, outputs)


# speedup vs baseline: 1.1865x; 1.0021x over previous
import functools

import jax
import jax.numpy as jnp
from jax import lax
from jax.experimental import pallas as pl
from jax.experimental.pallas import tpu as pltpu
from jax.experimental.pallas import tpu_sc as plsc

_N_MARGIN = 0.3
_LANES = 16
_C0 = 0.6931473219368033
_C2 = 0.12499505481299149
_C4 = -0.005181712215175508
_C6 = 0.00030122702988250955


def _sc_geometry():
    try:
        info = plsc.get_sparse_core_info()
        return info.num_cores, info.num_subcores
    except Exception:
        return 2, 16


@functools.lru_cache(maxsize=None)
def _make_sc_kernel(Bn, Dn, NNEG):
    NC, NS = _sc_geometry()
    NW = NC * NS
    RW = Bn // NW
    NV = RW * NNEG
    CH = 128
    NCH = NV // CH
    RPC = CH // NNEG
    KD = Dn // _LANES
    mesh = plsc.VectorSubcoreMesh(core_axis_name="c", subcore_axis_name="s")

    assert NNEG == 2 * _LANES and RW % _LANES == 0

    @functools.partial(
        pl.kernel,
        out_type=(
            jax.ShapeDtypeStruct((NW * _LANES,), jnp.float32),
            jax.ShapeDtypeStruct((NW * _LANES,), jnp.float32),
        ),
        mesh=mesh,
        compiler_params=pltpu.CompilerParams(needs_layout_passes=False),
        scratch_types=[
            pltpu.VMEM((NV,), jnp.int32),
            pltpu.VMEM((NV,), jnp.int32),
            pltpu.VMEM((RW,), jnp.int32),
            pltpu.VMEM((RW, Dn), jnp.float32),
            pltpu.VMEM((RW, Dn), jnp.float32),
            pltpu.VMEM((CH, Dn), jnp.float32),
            pltpu.VMEM((CH, Dn), jnp.float32),
            pltpu.VMEM((RW,), jnp.float32),
            pltpu.VMEM((_LANES,), jnp.float32),
            pltpu.VMEM((_LANES,), jnp.float32),
            pltpu.SemaphoreType.DMA,
            pltpu.SemaphoreType.DMA,
        ],
    )
    def sc(neg_hbm, tgt_hbm, alc_hbm, inp_hbm, v_hbm,
           sums_hbm, cnts_hbm,
           np_v, cid_v, tgt_v, vt_v, in_v, vc0_v, vc1_v, thr_v,
           sa_v, ca_v, sem0, sem1):
        wid = lax.axis_index("s") * NC + lax.axis_index("c")
        base = wid * RW
        lanes = lax.iota(jnp.int32, _LANES)

        pltpu.sync_copy(neg_hbm.at[pl.ds(base * NNEG, NV)], np_v)
        pltpu.sync_copy(tgt_hbm.at[pl.ds(base, RW)], tgt_v)
        pltpu.sync_copy(inp_hbm.at[pl.ds(base, RW)], in_v)

        waits = []
        for c in range(NCH):
            waits.append(pltpu.async_copy(
                alc_hbm.at[np_v.at[pl.ds(c * CH, CH)]],
                cid_v.at[pl.ds(c * CH, CH)], sem0))
        waits.append(pltpu.async_copy(v_hbm.at[tgt_v], vt_v, sem0))
        for w in waits:
            w.wait()

        bufs = (vc0_v, vc1_v)
        sems = (sem0, sem1)
        cps = [None, None]
        cps[0] = pltpu.async_copy(
            v_hbm.at[cid_v.at[pl.ds(0, CH)]], bufs[0], sems[0])

        zvec = jnp.zeros((_LANES,), jnp.float32)
        for g in range(RW // _LANES):
            def rloop(rr, carry, g=g):
                viv, vss = carry
                r = g * _LANES + rr
                piv = []
                pss = []
                for k in range(KD):
                    xi = in_v[r, pl.ds(k * _LANES, _LANES)]
                    piv.append(xi * vt_v[r, pl.ds(k * _LANES, _LANES)])
                    pss.append(xi * xi)
                while len(piv) > 1:
                    piv = [piv[i] + piv[i + 1] for i in range(0, len(piv), 2)]
                    pss = [pss[i] + pss[i + 1] for i in range(0, len(pss), 2)]
                tiv = jnp.sum(piv[0])
                tss = jnp.sum(pss[0])
                return (jnp.where(lanes == rr, tiv, viv),
                        jnp.where(lanes == rr, tss, vss))
            viv, vss = lax.fori_loop(0, _LANES, rloop, (zvec, zvec))
            u = plsc.bitcast(vss, jnp.int32)
            y = plsc.bitcast(jnp.int32(0x5F3759DF) - (u >> 1), jnp.float32)
            for _ in range(3):
                y = y * (1.5 - 0.5 * vss * y * y)
            thr_v[pl.ds(g * _LANES, _LANES)] = viv * y - _N_MARGIN

        zmask = jnp.zeros((_LANES,), jnp.bool_)
        sacc = zvec
        cacc = zvec
        for c in range(NCH):
            pb = c % 2
            if c + 1 < NCH:
                cps[(c + 1) % 2] = pltpu.async_copy(
                    v_hbm.at[cid_v.at[pl.ds((c + 1) * CH, CH)]],
                    bufs[(c + 1) % 2], sems[(c + 1) % 2])
            cps[pb].wait()
            vc_v = bufs[pb]

            def rlbody(rl, carry, c=c, vc_v=vc_v):
                sacc, cacc = carry
                r = c * RPC + rl
                vt_regs = [vt_v[r, pl.ds(k * _LANES, _LANES)]
                           for k in range(KD)]
                row_c0 = cid_v[pl.ds(r * NNEG, _LANES)]
                row_c1 = cid_v[pl.ds(r * NNEG + _LANES, _LANES)]

                def jloop(j, carry2, r=r, vt_regs=vt_regs, rl=rl, vc_v=vc_v):
                    v0, v1 = carry2
                    d = rl * NNEG + j
                    parts = [vt_v[r, pl.ds(k * _LANES, _LANES)] *
                             vc_v[d, pl.ds(k * _LANES, _LANES)]
                             for k in range(KD)]
                    while len(parts) > 1:
                        parts = [parts[i] + parts[i + 1]
                                 for i in range(0, len(parts), 2)]
                    tot = jnp.sum(parts[0])
                    return (jnp.where(lanes == j, tot, v0),
                            jnp.where(lanes == (j - _LANES), tot, v1))
                v0, v1 = lax.fori_loop(0, NNEG, jloop, (zvec, zvec),
                                       unroll=2)

                def kloop(k, carry2, r=r, row_c0=row_c0, row_c1=row_c1):
                    d0, d1 = carry2
                    ckv = plsc.load_gather(
                        cid_v, [jnp.broadcast_to(r * NNEG + k, (_LANES,))])
                    d0 = d0 | ((row_c0 == ckv) & (lanes > k))
                    d1 = d1 | ((row_c1 == ckv) & ((lanes + _LANES) > k))
                    return d0, d1
                dup0, dup1 = lax.fori_loop(0, NNEG, kloop, (zmask, zmask),
                                           unroll=2)

                thrv = plsc.load_gather(thr_v, [jnp.broadcast_to(r,
                                                                 (_LANES,))])
                for vv, dd in ((v0, dup0), (v1, dup1)):
                    hard = (~dd) & (vv > thrv) & (vv < 0.999999)
                    t2 = vv * vv
                    sp = 0.5 * vv + (_C0 + t2 * (_C2 + t2 * (_C4 + t2 * _C6)))
                    sacc = sacc + jnp.where(hard, sp, 0.0)
                    cacc = cacc + jnp.where(hard, 1.0, 0.0)
                return sacc, cacc
            sacc, cacc = lax.fori_loop(0, RPC, rlbody, (sacc, cacc))

        sa_v[...] = sacc
        ca_v[...] = cacc
        pltpu.sync_copy(sa_v, sums_hbm.at[pl.ds(wid * _LANES, _LANES)])
        pltpu.sync_copy(ca_v, cnts_hbm.at[pl.ds(wid * _LANES, _LANES)])

    return sc


@functools.lru_cache(maxsize=None)
def _make_matmul(Bn, Dn, Cn):
    CBLK = 1024

    def mm(x_ref, v_ref, o_ref):
        o_ref[...] = lax.dot_general(
            x_ref[...], v_ref[...], (((1,), (1,)), ((), ())),
            preferred_element_type=jnp.float32)

    return pl.pallas_call(
        mm,
        grid=(Cn // CBLK,),
        in_specs=[pl.BlockSpec((Bn, Dn), lambda i: (0, 0)),
                  pl.BlockSpec((CBLK, Dn), lambda i: (i, 0))],
        out_specs=pl.BlockSpec((Bn, CBLK), lambda i: (0, i)),
        out_shape=jax.ShapeDtypeStruct((Bn, Cn), jnp.float32),
    )


def kernel(inputs, targets, pos_pairs, neg_pairs, indexs,
           all_label_to_clusterid, V):
    Bn, Dn = inputs.shape
    Cn = V.shape[0]
    NNEG = neg_pairs.shape[1]

    outputs = _make_matmul(Bn, Dn, Cn)(inputs, V)

    def _i32(x):
        return x if x.dtype == jnp.int32 else x.astype(jnp.int32)

    neg_flat = _i32(neg_pairs.reshape(-1))
    sums, cnts = _make_sc_kernel(Bn, Dn, NNEG)(
        neg_flat, _i32(targets), _i32(all_label_to_clusterid), inputs, V)

    s = jnp.sum(sums)
    c = jnp.sum(cnts)
    loss = jnp.where(c > 0.0, s / jnp.maximum(c, 1.0), jnp.float32(0.0))
    return (loss, outputs)
